# Initial kernel scaffold; baseline (speedup 1.0000x reference)
import functools

import jax
import jax.numpy as jnp
from jax import lax
from jax.experimental import pallas as pl
from jax.experimental.pallas import tpu as pltpu
from jax.experimental.pallas import tpu_sc as plsc

D_MODEL = 1024
PAD_ID = 0
SCALE = 32.0

NC = 2
NS = 16
NW = NC * NS
CHUNK = 64


def _prep_table(table):
    def body(t_ref, o_ref):
        rows = lax.broadcasted_iota(jnp.int32, t_ref.shape, 0)
        o_ref[...] = jnp.where(rows == PAD_ID, 0.0, t_ref[...] * SCALE)

    return pl.pallas_call(
        body,
        out_shape=jax.ShapeDtypeStruct(table.shape, table.dtype),
    )(table)


def _make_gather(V, D, B):
    assert B % (8 * NW) == 0
    b_per_w = B // NW
    assert b_per_w % (2 * CHUNK) == 0
    mesh = plsc.VectorSubcoreMesh(core_axis_name="c", subcore_axis_name="s")

    @functools.partial(
        pl.kernel,
        mesh=mesh,
        out_type=jax.ShapeDtypeStruct((B, D), jnp.float32),
        scratch_types=[
            pltpu.VMEM((b_per_w,), jnp.int32),
            pltpu.VMEM((CHUNK, D), jnp.float32),
            pltpu.VMEM((CHUNK, D), jnp.float32),
            pltpu.SemaphoreType.DMA,
            pltpu.SemaphoreType.DMA,
        ],
    )
    def gather_kernel(table_hbm, idx_hbm, out_hbm, idx_v, rows0, rows1, sem0, sem1):
        wid = lax.axis_index("s") * NC + lax.axis_index("c")
        base = wid * b_per_w
        pltpu.sync_copy(idx_hbm.at[pl.ds(base, b_per_w)], idx_v)

        pltpu.async_copy(table_hbm.at[idx_v.at[pl.ds(0, CHUNK)]], rows0, sem0)

        @pl.loop(0, b_per_w, step=2 * CHUNK)
        def _(c):
            pltpu.async_copy(
                table_hbm.at[idx_v.at[pl.ds(c + CHUNK, CHUNK)]], rows1, sem1
            )
            pltpu.make_async_copy(
                table_hbm.at[idx_v.at[pl.ds(c, CHUNK)]], rows0, sem0
            ).wait()
            pltpu.sync_copy(rows0, out_hbm.at[pl.ds(base + c, CHUNK)])

            @pl.when(c + 2 * CHUNK < b_per_w)
            def _():
                pltpu.async_copy(
                    table_hbm.at[idx_v.at[pl.ds(c + 2 * CHUNK, CHUNK)]], rows0, sem0
                )

            pltpu.make_async_copy(
                table_hbm.at[idx_v.at[pl.ds(c + CHUNK, CHUNK)]], rows1, sem1
            ).wait()
            pltpu.sync_copy(rows1, out_hbm.at[pl.ds(base + c + CHUNK, CHUNK)])

    return gather_kernel


def kernel(input_ids, table):
    B = input_ids.size
    V, D = table.shape
    scaled = _prep_table(table)
    ids = input_ids.reshape(B)
    out = _make_gather(V, D, B)(scaled, ids)
    return out.reshape(input_ids.shape + (D,))


# speedup vs baseline: 1.6291x; 1.6291x over previous
import functools

import jax
import jax.numpy as jnp
from jax import lax
from jax.experimental import pallas as pl
from jax.experimental.pallas import tpu as pltpu
from jax.experimental.pallas import tpu_sc as plsc

D_MODEL = 1024
PAD_ID = 0
SCALE = 32.0

NC = 2
NS = 16
NW = NC * NS
CHUNK = 32


def _prep_table(table):
    def body(t_ref, o_ref):
        rows = lax.broadcasted_iota(jnp.int32, t_ref.shape, 0)
        o_ref[...] = jnp.where(rows == PAD_ID, 0.0, t_ref[...] * SCALE)

    return pl.pallas_call(
        body,
        out_shape=jax.ShapeDtypeStruct(table.shape, table.dtype),
    )(table)


def _make_gather(V, D, B):
    assert B % (8 * NW) == 0
    b_per_w = B // NW
    assert b_per_w % (2 * CHUNK) == 0
    mesh = plsc.VectorSubcoreMesh(core_axis_name="c", subcore_axis_name="s")

    @functools.partial(
        pl.kernel,
        mesh=mesh,
        out_type=jax.ShapeDtypeStruct((B, D), jnp.float32),
        scratch_types=[
            pltpu.VMEM((b_per_w,), jnp.int32),
            pltpu.VMEM((CHUNK, D), jnp.float32),
            pltpu.VMEM((CHUNK, D), jnp.float32),
            pltpu.SemaphoreType.DMA,
            pltpu.SemaphoreType.DMA,
        ],
    )
    def gather_kernel(table_hbm, idx_hbm, out_hbm, idx_v, rows0, rows1, sem0, sem1):
        wid = lax.axis_index("s") * NC + lax.axis_index("c")
        base = wid * b_per_w
        pltpu.sync_copy(idx_hbm.at[pl.ds(base, b_per_w)], idx_v)

        pltpu.async_copy(table_hbm.at[idx_v.at[pl.ds(0, CHUNK)]], rows0, sem0)

        @pl.loop(0, b_per_w, step=2 * CHUNK)
        def _(c):
            pltpu.async_copy(
                table_hbm.at[idx_v.at[pl.ds(c + CHUNK, CHUNK)]], rows1, sem1
            )
            pltpu.make_async_copy(
                table_hbm.at[idx_v.at[pl.ds(c, CHUNK)]], rows0, sem0
            ).wait()
            pltpu.sync_copy(rows0, out_hbm.at[pl.ds(base + c, CHUNK)])

            @pl.when(c + 2 * CHUNK < b_per_w)
            def _():
                pltpu.async_copy(
                    table_hbm.at[idx_v.at[pl.ds(c + 2 * CHUNK, CHUNK)]], rows0, sem0
                )

            pltpu.make_async_copy(
                table_hbm.at[idx_v.at[pl.ds(c + CHUNK, CHUNK)]], rows1, sem1
            ).wait()
            pltpu.sync_copy(rows1, out_hbm.at[pl.ds(base + c + CHUNK, CHUNK)])

    return gather_kernel


def kernel(input_ids, table):
    B = input_ids.size
    V, D = table.shape
    scaled = _prep_table(table)
    ids = input_ids.reshape(B)
    out = _make_gather(V, D, B)(scaled, ids)
    return out.reshape(input_ids.shape + (D,))
